# baseline (speedup 1.0000x reference)
import functools

import jax
import jax.numpy as jnp
from jax import lax
from jax.experimental import pallas as pl
from jax.experimental.pallas import tpu as pltpu
from jax.experimental.pallas import tpu_sc as plsc

N = 10000
E = 320000
D = 128

NC = 2
NS = 16
NW = NC * NS

E_PER_W = E // NW
CH = 125
N_CH = E_PER_W // CH
IB = 16
NB = N_CH // IB
N_PAD = 10240
ROWS_PER_S = N_PAD // NS
ZR = 64
N_FLUSH = ROWS_PER_S // ZR

MM_BLK = 2000


def _mm_body(x_ref, w_ref, n_ref, o_ref):
    h = jnp.dot(x_ref[...], w_ref[...], preferred_element_type=jnp.float32)
    o_ref[...] = h * n_ref[...]


def _post_body(p_ref, n_ref, b_ref, o_ref):
    agg = p_ref[0] + p_ref[1]
    o_ref[...] = agg * n_ref[...] + b_ref[...]


def _run_block(h_hbm, acc, si, di, rows0, rows1, sem0, sem1, ss0, ss1):
    pltpu.async_copy(h_hbm.at[si.at[0]], rows0, sem0)
    pltpu.async_copy(h_hbm.at[si.at[1]], rows1, sem1)

    @pl.loop(0, IB // 2 - 1)
    def _(t):
        i = t * 2
        pltpu.make_async_copy(h_hbm.at[si.at[i]], rows0, sem0).wait()
        pltpu.async_copy(rows0, acc.at[di.at[i]], ss0, add=True)
        pltpu.make_async_copy(h_hbm.at[si.at[i + 1]], rows1, sem1).wait()
        pltpu.async_copy(rows1, acc.at[di.at[i + 1]], ss1, add=True)
        pltpu.make_async_copy(rows0, acc.at[di.at[i]], ss0).wait()
        pltpu.async_copy(h_hbm.at[si.at[i + 2]], rows0, sem0)
        pltpu.make_async_copy(rows1, acc.at[di.at[i + 1]], ss1).wait()
        pltpu.async_copy(h_hbm.at[si.at[i + 3]], rows1, sem1)

    pltpu.make_async_copy(h_hbm.at[si.at[IB - 2]], rows0, sem0).wait()
    pltpu.async_copy(rows0, acc.at[di.at[IB - 2]], ss0, add=True)
    pltpu.make_async_copy(h_hbm.at[si.at[IB - 1]], rows1, sem1).wait()
    pltpu.async_copy(rows1, acc.at[di.at[IB - 1]], ss1, add=True)
    pltpu.make_async_copy(rows0, acc.at[di.at[IB - 2]], ss0).wait()
    pltpu.make_async_copy(rows1, acc.at[di.at[IB - 1]], ss1).wait()


def _seg_body(h_hbm, src_hbm, dst_hbm, out_hbm, src_i0, dst_i0, src_i1, dst_i1,
              rows0, rows1, acc, sem0, sem1, isem0, isem1, ss0, ss1):
    c = lax.axis_index("core")
    s = lax.axis_index("subcore")
    w = c * NS + s

    zeros16 = jnp.zeros((16,), jnp.float32)

    @pl.loop(0, ZR)
    def _(i):
        @pl.loop(0, D // 16)
        def _(j):
            rows0[i, pl.ds(j * 16, 16)] = zeros16

    pltpu.sync_copy(src_hbm.at[w, pl.ds(0, IB)], src_i0)
    pltpu.sync_copy(dst_hbm.at[w, pl.ds(0, IB)], dst_i0)

    zsrc = rows0.at[pl.ds(0, ZR)]

    @pl.loop(0, N_FLUSH)
    def _(t):
        pltpu.async_copy(zsrc, acc.at[pl.ds(s * ROWS_PER_S + t * ZR, ZR)], sem1)

    @pl.loop(0, N_FLUSH)
    def _(t):
        pltpu.make_async_copy(zsrc, acc.at[pl.ds(s * ROWS_PER_S + t * ZR, ZR)], sem1).wait()

    plsc.subcore_barrier()

    sblk = [(src_i0, dst_i0, isem0), (src_i1, dst_i1, isem1)]
    for b in range(NB):
        si, di, _ = sblk[b % 2]
        if b + 1 < NB:
            nsi, ndi, nisem = sblk[(b + 1) % 2]
            pltpu.async_copy(src_hbm.at[w, pl.ds((b + 1) * IB, IB)], nsi, nisem)
            pltpu.async_copy(dst_hbm.at[w, pl.ds((b + 1) * IB, IB)], ndi, nisem)
        _run_block(h_hbm, acc, si, di, rows0, rows1, sem0, sem1, ss0, ss1)
        if b + 1 < NB:
            pltpu.make_async_copy(src_hbm.at[w, pl.ds((b + 1) * IB, IB)], nsi, nisem).wait()
            pltpu.make_async_copy(dst_hbm.at[w, pl.ds((b + 1) * IB, IB)], ndi, nisem).wait()

    plsc.subcore_barrier()

    for t in range(N_FLUSH):
        k = t % 2
        rbuf = (rows0 if k == 0 else rows1).at[pl.ds(0, ZR)]
        sem = sem0 if k == 0 else sem1
        r = s * ROWS_PER_S + t * ZR
        if t >= 2:
            rp = s * ROWS_PER_S + (t - 2) * ZR
            pltpu.make_async_copy(rbuf, out_hbm.at[pl.ds(c * N_PAD + rp, ZR)], sem).wait()
        pltpu.sync_copy(acc.at[pl.ds(r, ZR)], rbuf)
        pltpu.async_copy(rbuf, out_hbm.at[pl.ds(c * N_PAD + r, ZR)], sem)
    for t in range(N_FLUSH - 2, N_FLUSH):
        k = t % 2
        rbuf = (rows0 if k == 0 else rows1).at[pl.ds(0, ZR)]
        sem = sem0 if k == 0 else sem1
        r = s * ROWS_PER_S + t * ZR
        pltpu.make_async_copy(rbuf, out_hbm.at[pl.ds(c * N_PAD + r, ZR)], sem).wait()


def kernel(x, edge_index, norm, weight, bias):
    h = pl.pallas_call(
        _mm_body,
        grid=(N // MM_BLK,),
        in_specs=[
            pl.BlockSpec((MM_BLK, D), lambda i: (i, 0)),
            pl.BlockSpec((D, D), lambda i: (0, 0)),
            pl.BlockSpec((MM_BLK, 1), lambda i: (i, 0)),
        ],
        out_specs=pl.BlockSpec((MM_BLK, D), lambda i: (i, 0)),
        out_shape=jax.ShapeDtypeStruct((N, D), jnp.float32),
    )(x, weight, norm)

    src = edge_index[0].reshape(NW, N_CH, CH)
    dst = edge_index[1].reshape(NW, N_CH, CH)

    mesh = plsc.VectorSubcoreMesh(core_axis_name="core", subcore_axis_name="subcore")
    seg = functools.partial(
        pl.kernel,
        mesh=mesh,
        out_type=jax.ShapeDtypeStruct((NC * N_PAD, D), jnp.float32),
        scratch_types=[
            pltpu.VMEM((IB, CH), jnp.int32),
            pltpu.VMEM((IB, CH), jnp.int32),
            pltpu.VMEM((IB, CH), jnp.int32),
            pltpu.VMEM((IB, CH), jnp.int32),
            pltpu.VMEM((CH, D), jnp.float32),
            pltpu.VMEM((CH, D), jnp.float32),
            pltpu.VMEM_SHARED((N_PAD, D), jnp.float32),
            pltpu.SemaphoreType.DMA,
            pltpu.SemaphoreType.DMA,
            pltpu.SemaphoreType.DMA,
            pltpu.SemaphoreType.DMA,
            pltpu.SemaphoreType.DMA,
            pltpu.SemaphoreType.DMA,
        ],
    )(_seg_body)
    partial = seg(h, src, dst)
    partial = partial.reshape(NC, N_PAD, D)

    bias2d = bias.reshape(1, D)
    out = pl.pallas_call(
        _post_body,
        grid=(N // MM_BLK,),
        in_specs=[
            pl.BlockSpec((NC, MM_BLK, D), lambda i: (0, i, 0)),
            pl.BlockSpec((MM_BLK, 1), lambda i: (i, 0)),
            pl.BlockSpec((1, D), lambda i: (0, 0)),
        ],
        out_specs=pl.BlockSpec((MM_BLK, D), lambda i: (i, 0)),
        out_shape=jax.ShapeDtypeStruct((N, D), jnp.float32),
    )(partial, norm, bias2d)
    return out


# speedup vs baseline: 1.2305x; 1.2305x over previous
import functools

import jax
import jax.numpy as jnp
from jax import lax
from jax.experimental import pallas as pl
from jax.experimental.pallas import tpu as pltpu
from jax.experimental.pallas import tpu_sc as plsc

N = 10000
E = 320000
D = 128

NC = 2
NS = 16
NW = NC * NS

E_PER_W = E // NW
CH = 125
N_CH = E_PER_W // CH
IB = 16
NB = N_CH // IB
N_PAD = 10240
ROWS_PER_S = N_PAD // NS
ZR = 80
N_FLUSH = ROWS_PER_S // ZR

MM_BLK = 2000


def _mm_body(x_ref, w_ref, n_ref, o_ref):
    h = jnp.dot(x_ref[...], w_ref[...], preferred_element_type=jnp.float32)
    o_ref[...] = h * n_ref[...]


def _post_body(p_ref, n_ref, b_ref, o_ref):
    agg = p_ref[0] + p_ref[1]
    o_ref[...] = agg * n_ref[...] + b_ref[...]


def _seg_body(h_hbm, src_hbm, dst_hbm, out_hbm, src_i0, dst_i0, src_i1, dst_i1,
              rows0, rows1, acc, sem0, sem1, isem0, isem1):
    c = lax.axis_index("core")
    s = lax.axis_index("subcore")
    w = c * NS + s

    zeros16 = jnp.zeros((16,), jnp.float32)

    @pl.loop(0, ZR)
    def _(i):
        @pl.loop(0, D // 16)
        def _(j):
            rows0[i, pl.ds(j * 16, 16)] = zeros16

    pltpu.sync_copy(src_hbm.at[w, pl.ds(0, IB)], src_i0)
    pltpu.sync_copy(dst_hbm.at[w, pl.ds(0, IB)], dst_i0)

    zsrc = rows0.at[pl.ds(0, ZR)]

    @pl.loop(0, N_FLUSH)
    def _(t):
        pltpu.async_copy(zsrc, acc.at[pl.ds(s * ROWS_PER_S + t * ZR, ZR)], sem1)

    @pl.loop(0, N_FLUSH)
    def _(t):
        pltpu.make_async_copy(zsrc, acc.at[pl.ds(s * ROWS_PER_S + t * ZR, ZR)], sem1).wait()

    plsc.subcore_barrier()

    def g_start(idx_ref, j, rbuf, sem):
        pltpu.async_copy(h_hbm.at[idx_ref.at[j]], rbuf, sem)

    def g_wait(idx_ref, j, rbuf, sem):
        pltpu.make_async_copy(h_hbm.at[idx_ref.at[j]], rbuf, sem).wait()

    def sc_add(idx_ref, j, rbuf):
        pltpu.sync_copy(rbuf, acc.at[idx_ref.at[j]], add=True)

    sblk = [(src_i0, dst_i0, isem0), (src_i1, dst_i1, isem1)]
    g_start(src_i0, 0, rows0, sem0)
    g_start(src_i0, 1, rows1, sem1)
    for b in range(NB):
        si, di, _ = sblk[b % 2]
        last = b + 1 == NB
        if not last:
            nsi, ndi, nisem = sblk[(b + 1) % 2]
            pltpu.async_copy(src_hbm.at[w, pl.ds((b + 1) * IB, IB)], nsi, nisem)
            pltpu.async_copy(dst_hbm.at[w, pl.ds((b + 1) * IB, IB)], ndi, nisem)

        @pl.loop(0, IB // 2 - 1)
        def _(t):
            i = t * 2
            g_wait(si, i, rows0, sem0)
            sc_add(di, i, rows0)
            g_start(si, i + 2, rows0, sem0)
            g_wait(si, i + 1, rows1, sem1)
            sc_add(di, i + 1, rows1)
            g_start(si, i + 3, rows1, sem1)

        if not last:
            pltpu.make_async_copy(src_hbm.at[w, pl.ds((b + 1) * IB, IB)], nsi, nisem).wait()
            pltpu.make_async_copy(dst_hbm.at[w, pl.ds((b + 1) * IB, IB)], ndi, nisem).wait()
        g_wait(si, IB - 2, rows0, sem0)
        sc_add(di, IB - 2, rows0)
        if not last:
            g_start(nsi, 0, rows0, sem0)
        g_wait(si, IB - 1, rows1, sem1)
        sc_add(di, IB - 1, rows1)
        if not last:
            g_start(nsi, 1, rows1, sem1)

    plsc.subcore_barrier()

    for t in range(N_FLUSH):
        k = t % 2
        rbuf = (rows0 if k == 0 else rows1).at[pl.ds(0, ZR)]
        sem = sem0 if k == 0 else sem1
        r = s * ROWS_PER_S + t * ZR
        if t >= 2:
            rp = s * ROWS_PER_S + (t - 2) * ZR
            pltpu.make_async_copy(rbuf, out_hbm.at[pl.ds(c * N_PAD + rp, ZR)], sem).wait()
        pltpu.sync_copy(acc.at[pl.ds(r, ZR)], rbuf)
        pltpu.async_copy(rbuf, out_hbm.at[pl.ds(c * N_PAD + r, ZR)], sem)
    for t in range(N_FLUSH - 2, N_FLUSH):
        k = t % 2
        rbuf = (rows0 if k == 0 else rows1).at[pl.ds(0, ZR)]
        sem = sem0 if k == 0 else sem1
        r = s * ROWS_PER_S + t * ZR
        pltpu.make_async_copy(rbuf, out_hbm.at[pl.ds(c * N_PAD + r, ZR)], sem).wait()


def kernel(x, edge_index, norm, weight, bias):
    h = pl.pallas_call(
        _mm_body,
        grid=(N // MM_BLK,),
        in_specs=[
            pl.BlockSpec((MM_BLK, D), lambda i: (i, 0)),
            pl.BlockSpec((D, D), lambda i: (0, 0)),
            pl.BlockSpec((MM_BLK, 1), lambda i: (i, 0)),
        ],
        out_specs=pl.BlockSpec((MM_BLK, D), lambda i: (i, 0)),
        out_shape=jax.ShapeDtypeStruct((N, D), jnp.float32),
    )(x, weight, norm)

    src = edge_index[0].reshape(NW, N_CH, CH)
    dst = edge_index[1].reshape(NW, N_CH, CH)

    mesh = plsc.VectorSubcoreMesh(core_axis_name="core", subcore_axis_name="subcore")
    seg = functools.partial(
        pl.kernel,
        mesh=mesh,
        out_type=jax.ShapeDtypeStruct((NC * N_PAD, D), jnp.float32),
        scratch_types=[
            pltpu.VMEM((IB, CH), jnp.int32),
            pltpu.VMEM((IB, CH), jnp.int32),
            pltpu.VMEM((IB, CH), jnp.int32),
            pltpu.VMEM((IB, CH), jnp.int32),
            pltpu.VMEM((CH, D), jnp.float32),
            pltpu.VMEM((CH, D), jnp.float32),
            pltpu.VMEM_SHARED((N_PAD, D), jnp.float32),
            pltpu.SemaphoreType.DMA,
            pltpu.SemaphoreType.DMA,
            pltpu.SemaphoreType.DMA,
            pltpu.SemaphoreType.DMA,
        ],
    )(_seg_body)
    partial = seg(h, src, dst)
    partial = partial.reshape(NC, N_PAD, D)

    bias2d = bias.reshape(1, D)
    out = pl.pallas_call(
        _post_body,
        grid=(N // MM_BLK,),
        in_specs=[
            pl.BlockSpec((NC, MM_BLK, D), lambda i: (0, i, 0)),
            pl.BlockSpec((MM_BLK, 1), lambda i: (i, 0)),
            pl.BlockSpec((1, D), lambda i: (0, 0)),
        ],
        out_specs=pl.BlockSpec((MM_BLK, D), lambda i: (i, 0)),
        out_shape=jax.ShapeDtypeStruct((N, D), jnp.float32),
    )(partial, norm, bias2d)
    return out


# speedup vs baseline: 1.2590x; 1.0232x over previous
import functools

import jax
import jax.numpy as jnp
from jax import lax
from jax.experimental import pallas as pl
from jax.experimental.pallas import tpu as pltpu
from jax.experimental.pallas import tpu_sc as plsc

N = 10000
E = 320000
D = 128

NC = 2
NS = 16
NW = NC * NS

E_PER_W = E // NW
CH = 125
N_CH = E_PER_W // CH
IB = 16
NB = N_CH // IB
N_PAD = 10240
ROWS_PER_S = N_PAD // NS
ZR = 80
N_FLUSH = ROWS_PER_S // ZR
ZB = 40
N_ZERO = ROWS_PER_S // ZB

MM_BLK = 2000


def _mm_body(x_ref, w_ref, n_ref, o_ref):
    h = jnp.dot(x_ref[...], w_ref[...], preferred_element_type=jnp.float32)
    o_ref[...] = h * n_ref[...]


def _post_body(p_ref, n_ref, b_ref, o_ref):
    agg = p_ref[0] + p_ref[1]
    o_ref[...] = agg * n_ref[...] + b_ref[...]


def _seg_body(h_hbm, src_hbm, dst_hbm, out_hbm, src_i0, dst_i0, src_i1, dst_i1,
              rows0, rows1, zbuf, acc, sem0, sem1, isem0, isem1):
    c = lax.axis_index("core")
    s = lax.axis_index("subcore")
    w = c * NS + s

    pltpu.async_copy(src_hbm.at[w, pl.ds(0, IB)], src_i0, isem0)
    pltpu.async_copy(dst_hbm.at[w, pl.ds(0, IB)], dst_i0, isem0)

    zeros16 = jnp.zeros((16,), jnp.float32)

    @pl.loop(0, ZB)
    def _(i):
        @pl.loop(0, D // 16)
        def _(j):
            zbuf[i, pl.ds(j * 16, 16)] = zeros16

    @pl.loop(0, N_ZERO)
    def _(t):
        pltpu.async_copy(zbuf, acc.at[pl.ds(s * ROWS_PER_S + t * ZB, ZB)], isem1)

    pltpu.make_async_copy(src_hbm.at[w, pl.ds(0, IB)], src_i0, isem0).wait()
    pltpu.make_async_copy(dst_hbm.at[w, pl.ds(0, IB)], dst_i0, isem0).wait()
    pltpu.async_copy(h_hbm.at[src_i0.at[0]], rows0, sem0)
    pltpu.async_copy(h_hbm.at[src_i0.at[1]], rows1, sem1)

    @pl.loop(0, N_ZERO)
    def _(t):
        pltpu.make_async_copy(zbuf, acc.at[pl.ds(s * ROWS_PER_S + t * ZB, ZB)], isem1).wait()

    plsc.subcore_barrier()

    def g_start(idx_ref, j, rbuf, sem):
        pltpu.async_copy(h_hbm.at[idx_ref.at[j]], rbuf, sem)

    def g_wait(idx_ref, j, rbuf, sem):
        pltpu.make_async_copy(h_hbm.at[idx_ref.at[j]], rbuf, sem).wait()

    def sc_add(idx_ref, j, rbuf):
        pltpu.sync_copy(rbuf, acc.at[idx_ref.at[j]], add=True)

    sblk = [(src_i0, dst_i0, isem0), (src_i1, dst_i1, isem1)]
    for b in range(NB):
        si, di, _ = sblk[b % 2]
        last = b + 1 == NB
        if not last:
            nsi, ndi, nisem = sblk[(b + 1) % 2]
            pltpu.async_copy(src_hbm.at[w, pl.ds((b + 1) * IB, IB)], nsi, nisem)
            pltpu.async_copy(dst_hbm.at[w, pl.ds((b + 1) * IB, IB)], ndi, nisem)

        @pl.loop(0, IB // 2 - 1)
        def _(t):
            i = t * 2
            g_wait(si, i, rows0, sem0)
            sc_add(di, i, rows0)
            g_start(si, i + 2, rows0, sem0)
            g_wait(si, i + 1, rows1, sem1)
            sc_add(di, i + 1, rows1)
            g_start(si, i + 3, rows1, sem1)

        if not last:
            pltpu.make_async_copy(src_hbm.at[w, pl.ds((b + 1) * IB, IB)], nsi, nisem).wait()
            pltpu.make_async_copy(dst_hbm.at[w, pl.ds((b + 1) * IB, IB)], ndi, nisem).wait()
        g_wait(si, IB - 2, rows0, sem0)
        sc_add(di, IB - 2, rows0)
        if not last:
            g_start(nsi, 0, rows0, sem0)
        g_wait(si, IB - 1, rows1, sem1)
        sc_add(di, IB - 1, rows1)
        if not last:
            g_start(nsi, 1, rows1, sem1)

    plsc.subcore_barrier()

    for t in range(N_FLUSH):
        k = t % 2
        rbuf = (rows0 if k == 0 else rows1).at[pl.ds(0, ZR)]
        sem = sem0 if k == 0 else sem1
        r = s * ROWS_PER_S + t * ZR
        if t >= 2:
            rp = s * ROWS_PER_S + (t - 2) * ZR
            pltpu.make_async_copy(rbuf, out_hbm.at[pl.ds(c * N_PAD + rp, ZR)], sem).wait()
        pltpu.sync_copy(acc.at[pl.ds(r, ZR)], rbuf)
        pltpu.async_copy(rbuf, out_hbm.at[pl.ds(c * N_PAD + r, ZR)], sem)
    for t in range(N_FLUSH - 2, N_FLUSH):
        k = t % 2
        rbuf = (rows0 if k == 0 else rows1).at[pl.ds(0, ZR)]
        sem = sem0 if k == 0 else sem1
        r = s * ROWS_PER_S + t * ZR
        pltpu.make_async_copy(rbuf, out_hbm.at[pl.ds(c * N_PAD + r, ZR)], sem).wait()


def kernel(x, edge_index, norm, weight, bias):
    h = pl.pallas_call(
        _mm_body,
        grid=(N // MM_BLK,),
        in_specs=[
            pl.BlockSpec((MM_BLK, D), lambda i: (i, 0)),
            pl.BlockSpec((D, D), lambda i: (0, 0)),
            pl.BlockSpec((MM_BLK, 1), lambda i: (i, 0)),
        ],
        out_specs=pl.BlockSpec((MM_BLK, D), lambda i: (i, 0)),
        out_shape=jax.ShapeDtypeStruct((N, D), jnp.float32),
    )(x, weight, norm)

    src = edge_index[0].reshape(NW, N_CH, CH)
    dst = edge_index[1].reshape(NW, N_CH, CH)

    mesh = plsc.VectorSubcoreMesh(core_axis_name="core", subcore_axis_name="subcore")
    seg = functools.partial(
        pl.kernel,
        mesh=mesh,
        out_type=jax.ShapeDtypeStruct((NC * N_PAD, D), jnp.float32),
        scratch_types=[
            pltpu.VMEM((IB, CH), jnp.int32),
            pltpu.VMEM((IB, CH), jnp.int32),
            pltpu.VMEM((IB, CH), jnp.int32),
            pltpu.VMEM((IB, CH), jnp.int32),
            pltpu.VMEM((CH, D), jnp.float32),
            pltpu.VMEM((CH, D), jnp.float32),
            pltpu.VMEM((ZB, D), jnp.float32),
            pltpu.VMEM_SHARED((N_PAD, D), jnp.float32),
            pltpu.SemaphoreType.DMA,
            pltpu.SemaphoreType.DMA,
            pltpu.SemaphoreType.DMA,
            pltpu.SemaphoreType.DMA,
        ],
    )(_seg_body)
    partial = seg(h, src, dst)
    partial = partial.reshape(NC, N_PAD, D)

    bias2d = bias.reshape(1, D)
    out = pl.pallas_call(
        _post_body,
        grid=(N // MM_BLK,),
        in_specs=[
            pl.BlockSpec((NC, MM_BLK, D), lambda i: (0, i, 0)),
            pl.BlockSpec((MM_BLK, 1), lambda i: (i, 0)),
            pl.BlockSpec((1, D), lambda i: (0, 0)),
        ],
        out_specs=pl.BlockSpec((MM_BLK, D), lambda i: (i, 0)),
        out_shape=jax.ShapeDtypeStruct((N, D), jnp.float32),
    )(partial, norm, bias2d)
    return out
